# baseline (speedup 1.0000x reference)
import jax
import jax.numpy as jnp
import numpy as np
from jax.experimental import pallas as pl
from jax.experimental.pallas import tpu as pltpu

_N, _C, _H, _W = 8, 96, 224, 224
_CB = 2

_DN = (((1,), (0,)), ((), ()))


def _group_ones(g):
    i = np.arange(_W)
    m = (i[:, None] // g) == (i[None, :] // g)
    return jnp.asarray(m, dtype=jnp.bfloat16)


def _wsum(x, m_ref):
    shape = x.shape
    x2 = x.reshape(-1, shape[-1])
    xh = x2.astype(jnp.bfloat16)
    xl = (x2 - xh.astype(jnp.float32)).astype(jnp.bfloat16)
    m = m_ref[...]
    s = jax.lax.dot_general(xh, m, _DN, preferred_element_type=jnp.float32)
    s = s + jax.lax.dot_general(xl, m, _DN,
                                preferred_element_type=jnp.float32)
    return s.reshape(shape)


def _rowsum(x, dist):
    ax = x.ndim - 2
    fwd = jnp.roll(x, -dist, axis=ax)
    bwd = jnp.roll(x, dist, axis=ax)
    mshape = tuple(x.shape[i] if i == ax else 1 for i in range(x.ndim))
    idx = jax.lax.broadcasted_iota(jnp.int32, mshape, ax)
    take_fwd = (idx // dist) % 2 == 0
    return x + jnp.where(take_fwd, fwd, bwd)


def _mask22(x, m_ref):
    s = _rowsum(_wsum(x, m_ref), 1)
    return jnp.where(s > 0, x, jnp.zeros_like(x))


def _mask44(x, m_ref):
    s = _rowsum(_rowsum(_wsum(x, m_ref), 1), 2)
    return jnp.where(s > 0, x, jnp.zeros_like(x))


def _mask12(x, m_ref):
    s = _wsum(x, m_ref)
    return jnp.where(s > 0, x, jnp.zeros_like(x))


def _stage(fn, c0, nch, x5, y, m):
    blk = (_N, _CB, _H // 8, 8, _W)
    off = c0 // _CB

    def xmap(c):
        return (0, c + off, 0, 0, 0)

    in_specs = [pl.BlockSpec(blk, xmap)]
    args = [x5]
    aliases = {}
    if y is not None:
        in_specs.append(pl.BlockSpec((1, 1, 1, 8, _W),
                                     lambda c: (0, 0, 0, 0, 0)))
        args.append(y)
        aliases = {1: 0}
    if m is not None:
        in_specs.append(pl.BlockSpec((_W, _W), lambda c: (0, 0)))
        args.append(m)

    def body(*refs):
        x_ref = refs[0]
        o_ref = refs[-1]
        if m is not None:
            o_ref[...] = fn(x_ref[...], refs[-2])
        else:
            o_ref[...] = x_ref[...]

    return pl.pallas_call(
        body,
        grid=(nch // _CB,),
        in_specs=in_specs,
        out_specs=pl.BlockSpec(blk, xmap),
        out_shape=jax.ShapeDtypeStruct((_N, _C, _H // 8, 8, _W),
                                       jnp.float32),
        input_output_aliases=aliases,
    )(*args)


def kernel(activation):
    x5 = activation.reshape(_N, _C, _H // 8, 8, _W)
    y = _stage(_mask22, 0, 32, x5, None, _group_ones(2))
    y = _stage(_mask44, 32, 32, x5, y, _group_ones(4))
    y = _stage(_mask12, 64, 16, x5, y, _group_ones(2))
    y = _stage(None, 80, 16, x5, y, None)
    return y.reshape(_N, _C, _H, _W)


# speedup vs baseline: 1.0747x; 1.0747x over previous
import jax
import jax.numpy as jnp
import numpy as np
from jax.experimental import pallas as pl
from jax.experimental.pallas import tpu as pltpu

_N, _C, _H, _W = 8, 96, 224, 224
_CB = 8

_DN = (((1,), (0,)), ((), ()))


def _group_ones(g):
    i = np.arange(_W)
    m = (i[:, None] // g) == (i[None, :] // g)
    return jnp.asarray(m, dtype=jnp.bfloat16)


def _wsum(x, m_ref):
    shape = x.shape
    x2 = x.reshape(-1, shape[-1])
    xh = x2.astype(jnp.bfloat16)
    xl = (x2 - xh.astype(jnp.float32)).astype(jnp.bfloat16)
    m = m_ref[...]
    s = jax.lax.dot_general(xh, m, _DN, preferred_element_type=jnp.float32)
    s = s + jax.lax.dot_general(xl, m, _DN,
                                preferred_element_type=jnp.float32)
    return s.reshape(shape)


def _rowsum(x, dist):
    ax = x.ndim - 2
    fwd = jnp.roll(x, -dist, axis=ax)
    bwd = jnp.roll(x, dist, axis=ax)
    mshape = tuple(x.shape[i] if i == ax else 1 for i in range(x.ndim))
    idx = jax.lax.broadcasted_iota(jnp.int32, mshape, ax)
    take_fwd = (idx // dist) % 2 == 0
    return x + jnp.where(take_fwd, fwd, bwd)


def _mask22(x, m_ref):
    s = _rowsum(_wsum(x, m_ref), 1)
    return jnp.where(s > 0, x, jnp.zeros_like(x))


def _mask44(x, m_ref):
    s = _rowsum(_rowsum(_wsum(x, m_ref), 1), 2)
    return jnp.where(s > 0, x, jnp.zeros_like(x))


def _mask12(x, m_ref):
    s = _wsum(x, m_ref)
    return jnp.where(s > 0, x, jnp.zeros_like(x))


def _stage(fn, c0, nch, x5, y, m):
    blk = (_N, _CB, _H // 8, 8, _W)
    off = c0 // _CB

    def xmap(c):
        return (0, c + off, 0, 0, 0)

    in_specs = [pl.BlockSpec(blk, xmap)]
    args = [x5]
    aliases = {}
    if y is not None:
        in_specs.append(pl.BlockSpec((1, 1, 1, 8, _W),
                                     lambda c: (0, 0, 0, 0, 0)))
        args.append(y)
        aliases = {1: 0}
    if m is not None:
        in_specs.append(pl.BlockSpec((_W, _W), lambda c: (0, 0)))
        args.append(m)

    def body(*refs):
        x_ref = refs[0]
        o_ref = refs[-1]
        if m is not None:
            o_ref[...] = fn(x_ref[...], refs[-2])
        else:
            o_ref[...] = x_ref[...]

    return pl.pallas_call(
        body,
        grid=(nch // _CB,),
        in_specs=in_specs,
        out_specs=pl.BlockSpec(blk, xmap),
        out_shape=jax.ShapeDtypeStruct((_N, _C, _H // 8, 8, _W),
                                       jnp.float32),
        input_output_aliases=aliases,
    )(*args)


def kernel(activation):
    x5 = activation.reshape(_N, _C, _H // 8, 8, _W)
    y = _stage(_mask22, 0, 32, x5, None, _group_ones(2))
    y = _stage(_mask44, 32, 32, x5, y, _group_ones(4))
    y = _stage(_mask12, 64, 16, x5, y, _group_ones(2))
    y = _stage(None, 80, 16, x5, y, None)
    return y.reshape(_N, _C, _H, _W)
